# baseline (speedup 1.0000x reference)
import functools

import jax
import jax.numpy as jnp
from jax import lax
from jax.experimental import pallas as pl
from jax.experimental.pallas import tpu as pltpu
from jax.experimental.pallas import tpu_sc as plsc

B = 4096
D = 8192
L = 16
NC = 2
NS = 16
NW = NC * NS
BPW = B // NW
NSLOT = 4
UNROLL = 4

_mesh = plsc.VectorSubcoreMesh(core_axis_name="c", subcore_axis_name="s")


def _row_dot(a_ref, w_ref):
    def inner(j, accs):
        base = j * (4 * UNROLL * L)
        new = []
        for q in range(4):
            acc = accs[q]
            for u in range(UNROLL):
                off = base + (q * UNROLL + u) * L
                acc = acc + a_ref[0, pl.ds(off, L)] * w_ref[0, pl.ds(off, L)]
            new.append(acc)
        return tuple(new)

    zeros = jnp.zeros((L,), jnp.float32)
    accs = lax.fori_loop(0, D // (4 * UNROLL * L), inner,
                         (zeros, zeros, zeros, zeros))
    return (accs[0] + accs[1]) + (accs[2] + accs[3])


@functools.partial(
    pl.kernel,
    mesh=_mesh,
    out_type=jax.ShapeDtypeStruct((B,), jnp.float32),
    compiler_params=pltpu.CompilerParams(needs_layout_passes=False),
    scratch_types=[
        pltpu.VMEM((BPW, 1), jnp.int32),
        pltpu.VMEM((BPW, 1), jnp.int32),
        [pltpu.VMEM((1, D), jnp.float32) for _ in range(NSLOT)],
        [pltpu.VMEM((1, D), jnp.float32) for _ in range(NSLOT)],
        pltpu.VMEM((BPW,), jnp.float32),
        pltpu.VMEM((L, L), jnp.float32),
        [pltpu.SemaphoreType.DMA for _ in range(NSLOT)],
        [pltpu.SemaphoreType.DMA for _ in range(NSLOT)],
    ],
)
def _slim_body(user_hbm, item_hbm, a_hbm, w_hbm, out_hbm,
               uidx, iidx, a_bufs, w_bufs, res, acc_buf, sems_a, sems_w):
    wid = lax.axis_index("s") * NC + lax.axis_index("c")
    pltpu.sync_copy(user_hbm.at[pl.ds(wid * BPW, BPW)], uidx)
    pltpu.sync_copy(item_hbm.at[pl.ds(wid * BPW, BPW)], iidx)

    lane_iota = lax.iota(jnp.int32, L)

    def start(e, slot):
        pltpu.async_copy(a_hbm.at[uidx.at[e]], a_bufs[slot], sems_a[slot])
        pltpu.async_copy(w_hbm.at[iidx.at[e]], w_bufs[slot], sems_w[slot])

    def wait(e, slot):
        pltpu.make_async_copy(
            a_hbm.at[uidx.at[e]], a_bufs[slot], sems_a[slot]).wait()
        pltpu.make_async_copy(
            w_hbm.at[iidx.at[e]], w_bufs[slot], sems_w[slot]).wait()

    for s in range(NSLOT - 1):
        start(s, s)

    def group_body(g, carry):
        for c in range(L):
            slot = c % NSLOT
            e = g * L + c
            wait(e, slot)

            @pl.when(e + NSLOT - 1 < BPW)
            def _():
                start(e + NSLOT - 1, (c + NSLOT - 1) % NSLOT)

            acc_buf[c] = _row_dot(a_bufs[slot], w_bufs[slot])

        totals = jnp.zeros((L,), jnp.float32)
        for c in range(L):
            col_idx = jnp.full((L,), c, jnp.int32)
            totals = totals + plsc.load_gather(acc_buf, [lane_iota, col_idx])
        res[pl.ds(g * L, L)] = totals
        return carry

    lax.fori_loop(0, BPW // L, group_body, 0)
    pltpu.sync_copy(res, out_hbm.at[pl.ds(wid * BPW, BPW)])


def kernel(user, item, A, W):
    user2 = user.astype(jnp.int32).reshape(B, 1)
    item2 = item.astype(jnp.int32).reshape(B, 1)
    return _slim_body(user2, item2, A, W)
